# baseline (speedup 1.0000x reference)
import functools

import jax
import jax.numpy as jnp
from jax import lax
from jax.experimental import pallas as pl
from jax.experimental.pallas import tpu as pltpu
from jax.experimental.pallas import tpu_sc as plsc

T_MAX = 1000
COND_DIM = 128
BATCH = 16384

NC = 2
NS = 16
LANES = 16
NW = NC * NS

TC_ROWS = 4096
SC_ROWS = BATCH - TC_ROWS
B_PER_W = SC_ROWS // NW
CHUNK = 128
N_CHUNKS = B_PER_W // CHUNK

STAGE_ROWS = 64

TC_BLOCK = 512
TC_BLOCKS = TC_ROWS // TC_BLOCK


def _sc_body(t_hbm, table_hbm, out_hbm, idx_flat, idx2, rows, tbl_sp, *sems):
    gsems = sems[:N_CHUNKS]
    wsem = sems[N_CHUNKS]
    core = lax.axis_index("c")
    s = lax.axis_index("s")
    wid = s * NC + core
    base = wid * B_PER_W

    row0 = jnp.where(s == NS - 1, T_MAX - STAGE_ROWS, s * STAGE_ROWS)
    stage1 = pltpu.async_copy(
        table_hbm.at[pl.ds(row0, STAGE_ROWS)], rows.at[pl.ds(0, STAGE_ROWS)], wsem
    )

    pltpu.sync_copy(t_hbm.at[pl.ds(base, B_PER_W)], idx_flat)
    stage1.wait()
    stage2 = pltpu.async_copy(
        rows.at[pl.ds(0, STAGE_ROWS)], tbl_sp.at[pl.ds(row0, STAGE_ROWS)], wsem
    )

    def adjust(i, _):
        v = idx_flat[pl.ds(i * LANES, LANES)]
        v = jnp.where(v == 0, T_MAX - 1, v - 1)
        idx2[i // (CHUNK // LANES), pl.ds((i % (CHUNK // LANES)) * LANES, LANES)] = v
        return _

    lax.fori_loop(0, B_PER_W // LANES, adjust, 0, unroll=False)
    stage2.wait()

    plsc.subcore_barrier()

    gathers = [
        pltpu.async_copy(
            tbl_sp.at[idx2.at[j]], rows.at[pl.ds(j * CHUNK, CHUNK)], gsems[j]
        )
        for j in range(N_CHUNKS)
    ]
    for j in range(N_CHUNKS):
        gathers[j].wait()
        pltpu.async_copy(
            rows.at[pl.ds(j * CHUNK, CHUNK)],
            out_hbm.at[pl.ds(base + j * CHUNK, CHUNK)],
            wsem,
        )

    pltpu.make_async_copy(out_hbm.at[pl.ds(0, B_PER_W)], rows, wsem).wait()


def _tc_body(t_ref, table_ref, out_ref):
    idx = t_ref[0, 0, :]
    idx = jnp.where(idx == 0, T_MAX - 1, idx - 1)
    onehot = (
        lax.broadcasted_iota(jnp.int32, (TC_BLOCK, T_MAX), 1) == idx[:, None]
    ).astype(jnp.float32)
    out_ref[...] = jax.lax.dot_general(
        onehot,
        table_ref[...],
        (((1,), (0,)), ((), ())),
        preferred_element_type=jnp.float32,
    )


SC_BLOCKS = SC_ROWS // TC_BLOCK


def _tc_gather(t, table):
    t3 = t.reshape(BATCH // TC_BLOCK, 1, TC_BLOCK)
    return pl.pallas_call(
        _tc_body,
        grid=(TC_BLOCKS,),
        in_specs=[
            pl.BlockSpec((1, 1, TC_BLOCK), lambda i: (i + SC_BLOCKS, 0, 0)),
            pl.BlockSpec((T_MAX, COND_DIM), lambda i: (0, 0)),
        ],
        out_specs=pl.BlockSpec((TC_BLOCK, COND_DIM), lambda i: (i + SC_BLOCKS, 0)),
        out_shape=jax.ShapeDtypeStruct((BATCH, COND_DIM), jnp.float32),
    )(t3, table)


@jax.jit
def kernel(t, time_emb):
    mesh = plsc.VectorSubcoreMesh(
        core_axis_name="c", subcore_axis_name="s", num_cores=NC, num_subcores=NS
    )
    run = pl.kernel(
        _sc_body,
        out_type=(),
        mesh=mesh,
        scratch_types=[
            pltpu.VMEM((B_PER_W,), jnp.int32),
            pltpu.VMEM((N_CHUNKS, CHUNK), jnp.int32),
            pltpu.VMEM((B_PER_W, COND_DIM), jnp.float32),
            pltpu.VMEM_SHARED((T_MAX, COND_DIM), jnp.float32),
        ]
        + [pltpu.SemaphoreType.DMA] * (N_CHUNKS + 1),
    )
    buf = jax.new_ref(_tc_gather(t, time_emb))
    run(t, time_emb, buf)
    return jax.freeze(buf)


# speedup vs baseline: 1.1362x; 1.1362x over previous
import jax
import jax.numpy as jnp
from jax import lax
from jax.experimental import pallas as pl
from jax.experimental.pallas import tpu as pltpu
from jax.experimental.pallas import tpu_sc as plsc

T_MAX = 1000
COND_DIM = 128
BATCH = 16384

NC = 2
NS = 16
LANES = 16
NW = NC * NS
B_PER_W = BATCH // NW
CHUNK = 128
N_CHUNKS = B_PER_W // CHUNK

STAGE_ROWS = 64
LAST_ROWS = T_MAX - (NS - 1) * STAGE_ROWS


def _emb_lookup_body(
    t_hbm, table_hbm, out_hbm, idx_flat, idx2, rows, tbl_sp, gs0, gs1, gs2, gs3, wsem
):
    gsems = [gs0, gs1, gs2, gs3]
    core = lax.axis_index("c")
    s = lax.axis_index("s")
    wid = s * NC + core
    base = wid * B_PER_W

    row0 = jnp.where(s == NS - 1, T_MAX - STAGE_ROWS, s * STAGE_ROWS)
    stage1 = pltpu.async_copy(
        table_hbm.at[pl.ds(row0, STAGE_ROWS)], rows.at[pl.ds(0, STAGE_ROWS)], wsem
    )

    pltpu.sync_copy(t_hbm.at[pl.ds(base, B_PER_W)], idx_flat)
    stage1.wait()
    stage2 = pltpu.async_copy(
        rows.at[pl.ds(0, STAGE_ROWS)], tbl_sp.at[pl.ds(row0, STAGE_ROWS)], wsem
    )

    def adjust(i, _):
        v = idx_flat[pl.ds(i * LANES, LANES)]
        v = jnp.where(v == 0, T_MAX - 1, v - 1)
        idx2[i // (CHUNK // LANES), pl.ds((i % (CHUNK // LANES)) * LANES, LANES)] = v
        return _

    lax.fori_loop(0, B_PER_W // LANES, adjust, 0, unroll=False)
    stage2.wait()

    plsc.subcore_barrier()

    gathers = [
        pltpu.async_copy(
            tbl_sp.at[idx2.at[j]], rows.at[pl.ds(j * CHUNK, CHUNK)], gsems[j]
        )
        for j in range(N_CHUNKS)
    ]
    for j in range(N_CHUNKS):
        gathers[j].wait()
        pltpu.async_copy(
            rows.at[pl.ds(j * CHUNK, CHUNK)],
            out_hbm.at[pl.ds(base + j * CHUNK, CHUNK)],
            wsem,
        )

    pltpu.make_async_copy(out_hbm.at[pl.ds(0, B_PER_W)], rows, wsem).wait()


@jax.jit
def kernel(t, time_emb):
    mesh = plsc.VectorSubcoreMesh(
        core_axis_name="c", subcore_axis_name="s", num_cores=NC, num_subcores=NS
    )
    run = pl.kernel(
        _emb_lookup_body,
        out_type=jax.ShapeDtypeStruct((BATCH, COND_DIM), jnp.float32),
        mesh=mesh,
        scratch_types=[
            pltpu.VMEM((B_PER_W,), jnp.int32),
            pltpu.VMEM((N_CHUNKS, CHUNK), jnp.int32),
            pltpu.VMEM((B_PER_W, COND_DIM), jnp.float32),
            pltpu.VMEM_SHARED((T_MAX, COND_DIM), jnp.float32),
            pltpu.SemaphoreType.DMA,
            pltpu.SemaphoreType.DMA,
            pltpu.SemaphoreType.DMA,
            pltpu.SemaphoreType.DMA,
            pltpu.SemaphoreType.DMA,
        ],
    )
    return run(t, time_emb)


# speedup vs baseline: 1.1366x; 1.0004x over previous
import jax
import jax.numpy as jnp
from jax import lax
from jax.experimental import pallas as pl
from jax.experimental.pallas import tpu as pltpu
from jax.experimental.pallas import tpu_sc as plsc

T_MAX = 1000
COND_DIM = 128
BATCH = 16384

NC = 2
NS = 16
LANES = 16
NW = NC * NS
B_PER_W = BATCH // NW
CHUNK = 128
N_CHUNKS = B_PER_W // CHUNK

STAGE_ROWS = 64


def _emb_lookup_body(
    t_hbm, table_hbm, out_hbm, idx_flat, idx2, rows, tbl_sp, gs0, gs1, gs2, gs3, wsem
):
    gsems = [gs0, gs1, gs2, gs3]
    core = lax.axis_index("c")
    s = lax.axis_index("s")
    wid = s * NC + core
    base = wid * B_PER_W

    row0 = jnp.where(s == NS - 1, T_MAX - STAGE_ROWS, s * STAGE_ROWS)
    stage1 = pltpu.async_copy(
        table_hbm.at[pl.ds(row0, STAGE_ROWS)], rows.at[pl.ds(0, STAGE_ROWS)], wsem
    )

    pltpu.sync_copy(t_hbm.at[pl.ds(base, B_PER_W)], idx_flat)
    stage1.wait()
    stage2 = pltpu.async_copy(
        rows.at[pl.ds(0, STAGE_ROWS)], tbl_sp.at[pl.ds(row0, STAGE_ROWS)], wsem
    )

    def adjust(i, _):
        v = idx_flat[pl.ds(i * LANES, LANES)]
        v = jnp.where(v == 0, T_MAX - 1, v - 1)
        idx2[i // (CHUNK // LANES), pl.ds((i % (CHUNK // LANES)) * LANES, LANES)] = v
        return _

    lax.fori_loop(0, B_PER_W // LANES, adjust, 0, unroll=False)
    stage2.wait()

    plsc.subcore_barrier()

    gathers = [
        pltpu.async_copy(
            tbl_sp.at[idx2.at[j]], rows.at[pl.ds(j * CHUNK, CHUNK)], gsems[j]
        )
        for j in range(N_CHUNKS)
    ]
    for j in range(N_CHUNKS):
        gathers[j].wait()
        pltpu.async_copy(
            rows.at[pl.ds(j * CHUNK, CHUNK)],
            out_hbm.at[pl.ds(base + j * CHUNK, CHUNK)],
            wsem,
        )

    pltpu.make_async_copy(out_hbm.at[pl.ds(0, B_PER_W)], rows, wsem).wait()


@jax.jit
def kernel(t, time_emb):
    mesh = plsc.VectorSubcoreMesh(
        core_axis_name="c", subcore_axis_name="s", num_cores=NC, num_subcores=NS
    )
    run = pl.kernel(
        _emb_lookup_body,
        out_type=jax.ShapeDtypeStruct((BATCH, COND_DIM), jnp.float32),
        mesh=mesh,
        scratch_types=[
            pltpu.VMEM((B_PER_W,), jnp.int32),
            pltpu.VMEM((N_CHUNKS, CHUNK), jnp.int32),
            pltpu.VMEM((B_PER_W, COND_DIM), jnp.float32),
            pltpu.VMEM_SHARED((T_MAX, COND_DIM), jnp.float32),
            pltpu.SemaphoreType.DMA,
            pltpu.SemaphoreType.DMA,
            pltpu.SemaphoreType.DMA,
            pltpu.SemaphoreType.DMA,
            pltpu.SemaphoreType.DMA,
        ],
    )
    return run(t, time_emb)
